# baseline (speedup 1.0000x reference)
import functools

import jax
import jax.numpy as jnp
import numpy as np
from jax import lax
from jax.experimental import pallas as pl
from jax.experimental.pallas import tpu as pltpu
from jax.experimental.pallas import tpu_sc as plsc

_DIM = 128
_BASE = 10000.0
_NC = 2
_NS = 16
_NW = _NC * _NS
_CHUNK = 2
_NBUF = 8


def _rope_tables(sq):
    freqs = 1.0 / (_BASE ** (np.arange(0, _DIM, 2, dtype=np.float32) / _DIM))
    f = np.outer(np.arange(sq, dtype=np.float32), freqs)
    cos = np.cos(f).astype(np.float32)
    sin = np.sin(f).astype(np.float32)
    c_tab = np.repeat(cos, 2, axis=1)
    s_tab = np.stack([-sin, sin], axis=-1).reshape(sq, _DIM)
    return c_tab.astype(np.float32), s_tab.astype(np.float32)


@jax.jit
def _sc_rope(query, key):
    sq, nb, nh, hh = query.shape
    n_pos_w = sq // _NW
    n_chunks = n_pos_w // _CHUNK

    c_tab, s_tab = _rope_tables(sq)
    mesh = plsc.VectorSubcoreMesh(core_axis_name="c", subcore_axis_name="s")

    @functools.partial(
        pl.kernel,
        mesh=mesh,
        out_type=[
            jax.ShapeDtypeStruct(query.shape, query.dtype),
            jax.ShapeDtypeStruct(key.shape, key.dtype),
        ],
        scratch_types=(
            [pltpu.VMEM((_CHUNK, nb, nh, hh), jnp.float32)] * _NBUF
            + [pltpu.VMEM((n_pos_w, _DIM), jnp.float32),
               pltpu.VMEM((n_pos_w, _DIM), jnp.float32)]
            + [pltpu.SemaphoreType.DMA] * (2 * _NBUF)
        ),
        compiler_params=pltpu.CompilerParams(needs_layout_passes=False),
    )
    def k(q_hbm, k_hbm, c_hbm, s_hbm, qo_hbm, ko_hbm, *rest):
        buf = rest[:_NBUF]
        ctab, stab = rest[_NBUF], rest[_NBUF + 1]
        sems_i = rest[_NBUF + 2:2 * _NBUF + 2]
        sems_o = rest[2 * _NBUF + 2:]
        swap = lax.iota(jnp.int32, 16) ^ 1
        offs = [swap + 16 * v for v in range(8)]
        wid = lax.axis_index("s") * _NC + lax.axis_index("c")
        wpos = wid * n_pos_w
        pltpu.sync_copy(c_hbm.at[pl.ds(wpos, n_pos_w)], ctab)
        pltpu.sync_copy(s_hbm.at[pl.ds(wpos, n_pos_w)], stab)

        def in_copy(src, ci, b, sem):
            return pltpu.make_async_copy(
                src.at[pl.ds(wpos + ci * _CHUNK, _CHUNK)], buf[b], sem)

        def out_copy(dst, ci, b, sem):
            return pltpu.make_async_copy(
                buf[b], dst.at[pl.ds(wpos + ci * _CHUNK, _CHUNK)], sem)

        def compute(b, ci):
            def pbody(p, carry):
                lp = ci * _CHUNK + p
                cs = [(ctab[lp, pl.ds(16 * v, 16)],
                       stab[lp, pl.ds(16 * v, 16)]) for v in range(8)]
                pv = jnp.full((16,), p, jnp.int32)

                @plsc.parallel_loop(0, nb * nh, unroll=4)
                def rbody(r):
                    b2 = r >> 4
                    h = r & 15
                    bv = jnp.full((16,), b2, jnp.int32)
                    hv = jnp.full((16,), h, jnp.int32)
                    xs = [buf[b][p, b2, h, pl.ds(16 * v, 16)] for v in range(8)]
                    xsws = [plsc.load_gather(buf[b], [pv, bv, hv, offs[v]])
                            for v in range(8)]
                    for v in range(8):
                        c, s = cs[v]
                        buf[b][p, b2, h, pl.ds(16 * v, 16)] = (
                            xs[v] * c + xsws[v] * s)

                return carry

            lax.fori_loop(0, _CHUNK, pbody, 0)

        def run(src, dst):
            in_copy(src, 0, 0, sems_i[0]).start()
            in_copy(src, 1, 1, sems_i[1]).start()

            def body(j, _):
                for u in range(_NBUF):
                    ci = _NBUF * j + u
                    fi = ci + 2
                    fb = (u + 2) % _NBUF

                    @pl.when(fi < n_chunks)
                    def _():
                        @pl.when(fi >= _NBUF)
                        def _():
                            out_copy(dst, fi - _NBUF, fb, sems_o[fb]).wait()

                        in_copy(src, fi, fb, sems_i[fb]).start()

                    in_copy(src, ci, u, sems_i[u]).wait()
                    compute(u, ci)
                    out_copy(dst, ci, u, sems_o[u]).start()
                return 0

            lax.fori_loop(0, n_chunks // _NBUF, body, 0)
            for t in range(_NBUF):
                ci = n_chunks - _NBUF + t
                out_copy(dst, ci, ci % _NBUF, sems_o[ci % _NBUF]).wait()

        run(q_hbm, qo_hbm)
        run(k_hbm, ko_hbm)

    qo, ko = k(query, key, c_tab, s_tab)
    return qo, ko


def kernel(query, key):
    return _sc_rope(query, key)


# speedup vs baseline: 1.0425x; 1.0425x over previous
import functools

import jax
import jax.numpy as jnp
import numpy as np
from jax import lax
from jax.experimental import pallas as pl
from jax.experimental.pallas import tpu as pltpu
from jax.experimental.pallas import tpu_sc as plsc

_DIM = 128
_BASE = 10000.0
_NC = 2
_NS = 16
_NW = _NC * _NS
_CHUNK = 4
_NBUF = 4


def _rope_tables(sq):
    freqs = 1.0 / (_BASE ** (np.arange(0, _DIM, 2, dtype=np.float32) / _DIM))
    f = np.outer(np.arange(sq, dtype=np.float32), freqs)
    cos = np.cos(f).astype(np.float32)
    sin = np.sin(f).astype(np.float32)
    c_tab = np.repeat(cos, 2, axis=1)
    s_tab = np.stack([-sin, sin], axis=-1).reshape(sq, _DIM)
    return c_tab.astype(np.float32), s_tab.astype(np.float32)


@jax.jit
def _sc_rope(query, key):
    sq, nb, nh, hh = query.shape
    n_pos_w = sq // _NW
    n_chunks = n_pos_w // _CHUNK

    c_tab, s_tab = _rope_tables(sq)
    mesh = plsc.VectorSubcoreMesh(core_axis_name="c", subcore_axis_name="s")

    @functools.partial(
        pl.kernel,
        mesh=mesh,
        out_type=[
            jax.ShapeDtypeStruct(query.shape, query.dtype),
            jax.ShapeDtypeStruct(key.shape, key.dtype),
        ],
        scratch_types=(
            [pltpu.VMEM((_CHUNK, nb, nh, hh), jnp.float32)] * _NBUF
            + [pltpu.VMEM((n_pos_w, _DIM), jnp.float32),
               pltpu.VMEM((n_pos_w, _DIM), jnp.float32)]
            + [pltpu.SemaphoreType.DMA] * (2 * _NBUF)
        ),
        compiler_params=pltpu.CompilerParams(needs_layout_passes=False),
    )
    def k(q_hbm, k_hbm, c_hbm, s_hbm, qo_hbm, ko_hbm, *rest):
        buf = rest[:_NBUF]
        ctab, stab = rest[_NBUF], rest[_NBUF + 1]
        sems_i = rest[_NBUF + 2:2 * _NBUF + 2]
        sems_o = rest[2 * _NBUF + 2:]
        swap = lax.iota(jnp.int32, 16) ^ 1
        offs = [swap + 16 * v for v in range(8)]
        wid = lax.axis_index("s") * _NC + lax.axis_index("c")
        wpos = wid * n_pos_w
        pltpu.sync_copy(c_hbm.at[pl.ds(wpos, n_pos_w)], ctab)
        pltpu.sync_copy(s_hbm.at[pl.ds(wpos, n_pos_w)], stab)

        def in_copy(src, ci, b, sem):
            return pltpu.make_async_copy(
                src.at[pl.ds(wpos + ci * _CHUNK, _CHUNK)], buf[b], sem)

        def out_copy(dst, ci, b, sem):
            return pltpu.make_async_copy(
                buf[b], dst.at[pl.ds(wpos + ci * _CHUNK, _CHUNK)], sem)

        def compute(b, ci):
            def pbody(p, carry):
                lp = ci * _CHUNK + p
                cs = [(ctab[lp, pl.ds(16 * v, 16)],
                       stab[lp, pl.ds(16 * v, 16)]) for v in range(8)]
                pv = jnp.full((16,), p, jnp.int32)

                @plsc.parallel_loop(0, nb * nh, unroll=4)
                def rbody(r):
                    b2 = r >> 4
                    h = r & 15
                    bv = jnp.full((16,), b2, jnp.int32)
                    hv = jnp.full((16,), h, jnp.int32)
                    xs = [buf[b][p, b2, h, pl.ds(16 * v, 16)] for v in range(8)]
                    xsws = [plsc.load_gather(buf[b], [pv, bv, hv, offs[v]])
                            for v in range(8)]
                    for v in range(8):
                        c, s = cs[v]
                        buf[b][p, b2, h, pl.ds(16 * v, 16)] = (
                            xs[v] * c + xsws[v] * s)

                return carry

            lax.fori_loop(0, _CHUNK, pbody, 0)

        def run(src, dst):
            in_copy(src, 0, 0, sems_i[0]).start()
            in_copy(src, 1, 1, sems_i[1]).start()

            def body(j, _):
                for u in range(_NBUF):
                    ci = _NBUF * j + u
                    fi = ci + 2
                    fb = (u + 2) % _NBUF

                    @pl.when(fi < n_chunks)
                    def _():
                        @pl.when(fi >= _NBUF)
                        def _():
                            out_copy(dst, fi - _NBUF, fb, sems_o[fb]).wait()

                        in_copy(src, fi, fb, sems_i[fb]).start()

                    in_copy(src, ci, u, sems_i[u]).wait()
                    compute(u, ci)
                    out_copy(dst, ci, u, sems_o[u]).start()
                return 0

            lax.fori_loop(0, n_chunks // _NBUF, body, 0)
            for t in range(_NBUF):
                ci = n_chunks - _NBUF + t
                out_copy(dst, ci, ci % _NBUF, sems_o[ci % _NBUF]).wait()

        run(q_hbm, qo_hbm)
        run(k_hbm, ko_hbm)

    qo, ko = k(query, key, c_tab, s_tab)
    return qo, ko


def kernel(query, key):
    return _sc_rope(query, key)
